# Initial kernel scaffold; baseline (speedup 1.0000x reference)
import jax
import jax.numpy as jnp
from jax.experimental import pallas as pl


def kernel(z_e, codebook):
    raise NotImplementedError("write your pallas kernel here")


# speedup vs baseline: 1.5885x; 1.5885x over previous
import jax
import jax.numpy as jnp
from jax import lax
from jax.experimental import pallas as pl

_NCODES = 1024
_LDIM = 32


def _vq_body(z_ref, cb_ref, ste_ref, idx_ref, delta_ref):
    z = z_ref[0]
    cb = cb_ref[...]
    t = z.shape[1]
    e2 = jnp.sum(cb * cb, axis=1)
    scores = lax.dot_general(
        cb, z, (((1,), (0,)), ((), ())),
        preferred_element_type=jnp.float32,
    )
    dist = e2[:, None] - 2.0 * scores
    idx = jnp.argmin(dist, axis=0).astype(jnp.int32)
    onehot = (
        lax.broadcasted_iota(jnp.int32, (_NCODES, t), 0) == idx[None, :]
    ).astype(jnp.float32)
    e_mat = lax.dot_general(
        cb, onehot, (((0,), (0,)), ((), ())),
        preferred_element_type=jnp.float32,
        precision=lax.Precision.HIGHEST,
    )
    ste_ref[0] = z + (e_mat - z)
    delta_ref[0] = z - e_mat
    idx_ref[0, 0] = idx


def kernel(z_e, codebook):
    b, d, h, w = z_e.shape
    t = h * w
    z2 = z_e.reshape(b, d, t)
    ste, idx, delta = pl.pallas_call(
        _vq_body,
        grid=(b,),
        in_specs=[
            pl.BlockSpec((1, d, t), lambda i: (i, 0, 0)),
            pl.BlockSpec((_NCODES, _LDIM), lambda i: (0, 0)),
        ],
        out_specs=[
            pl.BlockSpec((1, d, t), lambda i: (i, 0, 0)),
            pl.BlockSpec((1, 1, t), lambda i: (i, 0, 0)),
            pl.BlockSpec((1, d, t), lambda i: (i, 0, 0)),
        ],
        out_shape=[
            jax.ShapeDtypeStruct((b, d, t), jnp.float32),
            jax.ShapeDtypeStruct((b, 1, t), jnp.int32),
            jax.ShapeDtypeStruct((b, d, t), jnp.float32),
        ],
    )(z2, codebook)
    return (
        ste.reshape(b, d, h, w),
        idx.reshape(b, h, w),
        delta.reshape(b, d, h, w),
    )


# speedup vs baseline: 1.9933x; 1.2548x over previous
import functools

import jax
import jax.numpy as jnp
from jax import lax
from jax.experimental import pallas as pl
from jax.experimental.pallas import tpu as pltpu
from jax.experimental.pallas import tpu_sc as plsc

_NCODES = 1024
_LDIM = 32
_LANES = 16


def _argmin_body(z_ref, cb_ref, idx_ref):
    z = z_ref[0]
    cb = cb_ref[...]
    t = z.shape[1]
    e2 = jnp.sum(cb * cb, axis=1)
    scores = lax.dot_general(
        cb, z, (((1,), (0,)), ((), ())),
        preferred_element_type=jnp.float32,
    )
    dist = e2[:, None] - 2.0 * scores
    idx_ref[0, 0] = jnp.argmin(dist, axis=0).astype(jnp.int32)


def _gather_body(z_hbm, cbt_hbm, idx_hbm, ste_hbm, delta_hbm,
                 cbt_v, idx_v, z_v, ek_v):
    wid = lax.axis_index("s") * 2 + lax.axis_index("c")
    t = _NCODES
    pltpu.sync_copy(cbt_hbm, cbt_v)
    pltpu.sync_copy(idx_hbm.at[pl.ds(wid * t, t)], idx_v)
    pltpu.sync_copy(z_hbm.at[wid], z_v)

    def group(g, _):
        base = g * _LANES
        idx16 = idx_v[pl.ds(base, _LANES)]
        for d in range(_LDIM):
            row = jnp.full((_LANES,), d, jnp.int32)
            ek = plsc.load_gather(cbt_v, [row, idx16])
            ek_v[d, pl.ds(base, _LANES)] = ek
            z_v[d, pl.ds(base, _LANES)] = z_v[d, pl.ds(base, _LANES)] - ek
        return ()

    lax.fori_loop(0, t // _LANES, group, (), unroll=2)
    pltpu.sync_copy(ek_v, ste_hbm.at[wid])
    pltpu.sync_copy(z_v, delta_hbm.at[wid])


def kernel(z_e, codebook):
    b, d, h, w = z_e.shape
    t = h * w
    z2 = z_e.reshape(b, d, t)
    idx = pl.pallas_call(
        _argmin_body,
        grid=(b,),
        in_specs=[
            pl.BlockSpec((1, d, t), lambda i: (i, 0, 0)),
            pl.BlockSpec((_NCODES, _LDIM), lambda i: (0, 0)),
        ],
        out_specs=pl.BlockSpec((1, 1, t), lambda i: (i, 0, 0)),
        out_shape=jax.ShapeDtypeStruct((b, 1, t), jnp.int32),
    )(z2, codebook)

    cbt = codebook.T.reshape(d, _NCODES)
    idx_flat = idx.reshape(b * t)
    sc_gather = pl.kernel(
        _gather_body,
        mesh=plsc.VectorSubcoreMesh(core_axis_name="c", subcore_axis_name="s"),
        compiler_params=pltpu.CompilerParams(
            use_tc_tiling_on_sc=False, needs_layout_passes=False
        ),
        out_type=[
            jax.ShapeDtypeStruct((b, d, t), jnp.float32),
            jax.ShapeDtypeStruct((b, d, t), jnp.float32),
        ],
        scratch_types=[
            pltpu.VMEM((d, _NCODES), jnp.float32),
            pltpu.VMEM((t,), jnp.int32),
            pltpu.VMEM((d, t), jnp.float32),
            pltpu.VMEM((d, t), jnp.float32),
        ],
    )
    ste, delta = sc_gather(z2, cbt, idx_flat)
    return (
        ste.reshape(b, d, h, w),
        idx.reshape(b, h, w),
        delta.reshape(b, d, h, w),
    )
